# Initial kernel scaffold; baseline (speedup 1.0000x reference)
import jax
import jax.numpy as jnp
from jax.experimental import pallas as pl


def kernel(x, labels):
    raise NotImplementedError("write your pallas kernel here")


# speedup vs baseline: 6.3233x; 6.3233x over previous
import jax
import jax.numpy as jnp
from jax.experimental import pallas as pl
from jax.experimental.pallas import tpu as pltpu

_N = 320000
_D = 128
_K = 100
_KP = 128
_BLK = 3200
_NB = _N // _BLK


def _loss_kernel(x_ref, lab_ref, out_ref, sums_ref, counts_ref, centers_ref,
                 acc_ref):
    p = pl.program_id(0)
    i = pl.program_id(1)

    lab = lab_ref[0, 0, :]
    ids = jax.lax.broadcasted_iota(jnp.int32, (_BLK, _KP), 1)
    oh = (lab[:, None] == ids).astype(jnp.float32)
    x = x_ref[...]

    @pl.when(jnp.logical_and(p == 0, i == 0))
    def _init():
        sums_ref[...] = jnp.zeros_like(sums_ref)
        counts_ref[...] = jnp.zeros_like(counts_ref)

    @pl.when(p == 0)
    def _accum():
        sums_ref[...] += jax.lax.dot_general(
            oh, x, (((0,), (0,)), ((), ())),
            preferred_element_type=jnp.float32)
        counts_ref[...] += jnp.sum(oh, axis=0, keepdims=True)

    @pl.when(jnp.logical_and(p == 1, i == 0))
    def _centers_and_inter():
        cnt = counts_ref[0, :]
        inv = jnp.where(cnt > 0.0, 1.0 / cnt, 0.0)
        centers = sums_ref[...] * inv[:, None]
        centers_ref[...] = centers
        g = jax.lax.dot_general(
            centers, centers, (((1,), (1,)), ((), ())),
            preferred_element_type=jnp.float32,
            precision=jax.lax.Precision.HIGHEST)
        n2 = jnp.sum(centers * centers, axis=1)
        d2 = n2[:, None] + n2[None, :] - 2.0 * g
        r = jax.lax.broadcasted_iota(jnp.int32, (_KP, _KP), 0)
        c = jax.lax.broadcasted_iota(jnp.int32, (_KP, _KP), 1)
        valid = jnp.logical_and(r < c, c < _K)
        d = jnp.sqrt(jnp.maximum(d2, 0.0))
        acc_ref[0] = jnp.sum(jnp.where(valid, d, 0.0))
        acc_ref[1] = 0.0

    @pl.when(p == 1)
    def _intra():
        cg = jax.lax.dot_general(
            oh, centers_ref[...], (((1,), (0,)), ((), ())),
            preferred_element_type=jnp.float32)
        diff = x - cg
        acc_ref[1] += jnp.sum(jnp.sqrt(jnp.sum(diff * diff, axis=1)))

        @pl.when(i == _NB - 1)
        def _fin():
            inter = acc_ref[0]
            intra = acc_ref[1]
            out_ref[0, 0] = jnp.where(inter > 0.0, intra / inter, intra)


def kernel(x, labels):
    labels3 = labels.astype(jnp.int32).reshape(_NB, 1, _BLK)
    out = pl.pallas_call(
        _loss_kernel,
        grid=(2, _NB),
        in_specs=[
            pl.BlockSpec((_BLK, _D), lambda p, i: (i, 0)),
            pl.BlockSpec((1, 1, _BLK), lambda p, i: (i, 0, 0)),
        ],
        out_shape=jax.ShapeDtypeStruct((1, 1), jnp.float32),
        out_specs=pl.BlockSpec(memory_space=pltpu.SMEM),
        scratch_shapes=[
            pltpu.VMEM((_KP, _D), jnp.float32),
            pltpu.VMEM((1, _KP), jnp.float32),
            pltpu.VMEM((_KP, _D), jnp.float32),
            pltpu.SMEM((2,), jnp.float32),
        ],
    )(x, labels3)
    return out[0, 0]


# speedup vs baseline: 6.3751x; 1.0082x over previous
import jax
import jax.numpy as jnp
from jax.experimental import pallas as pl
from jax.experimental.pallas import tpu as pltpu

_N = 320000
_D = 128
_K = 100
_KP = 128
_BLK = 3200
_NB = _N // _BLK


def _loss_kernel(x_ref, lab_ref, out_ref, sums_ref, counts_ref, centers_ref,
                 acc_ref):
    p = pl.program_id(0)
    i = pl.program_id(1)

    lab = lab_ref[0, 0, :]
    ids = jax.lax.broadcasted_iota(jnp.int32, (_BLK, _KP), 1)
    oh = (lab[:, None] == ids).astype(jnp.float32)
    x = x_ref[...]

    @pl.when(jnp.logical_and(p == 0, i == 0))
    def _init():
        sums_ref[...] = jnp.zeros_like(sums_ref)
        counts_ref[...] = jnp.zeros_like(counts_ref)

    @pl.when(p == 0)
    def _accum():
        sums_ref[...] += jax.lax.dot_general(
            oh, x, (((0,), (0,)), ((), ())),
            preferred_element_type=jnp.float32)
        ones_row = jnp.ones((1, _BLK), jnp.float32)
        counts_ref[...] += jax.lax.dot_general(
            ones_row, oh, (((1,), (0,)), ((), ())),
            preferred_element_type=jnp.float32)

    @pl.when(jnp.logical_and(p == 1, i == 0))
    def _centers_and_inter():
        cnt = counts_ref[0, :]
        inv = jnp.where(cnt > 0.0, 1.0 / cnt, 0.0)
        centers = sums_ref[...] * inv[:, None]
        centers_ref[...] = centers
        g = jax.lax.dot_general(
            centers, centers, (((1,), (1,)), ((), ())),
            preferred_element_type=jnp.float32,
            precision=jax.lax.Precision.HIGHEST)
        n2 = jnp.sum(centers * centers, axis=1)
        d2 = n2[:, None] + n2[None, :] - 2.0 * g
        r = jax.lax.broadcasted_iota(jnp.int32, (_KP, _KP), 0)
        c = jax.lax.broadcasted_iota(jnp.int32, (_KP, _KP), 1)
        valid = jnp.logical_and(r < c, c < _K)
        d = jnp.sqrt(jnp.maximum(d2, 0.0))
        acc_ref[0] = jnp.sum(jnp.where(valid, d, 0.0))
        acc_ref[1] = 0.0

    @pl.when(p == 1)
    def _intra():
        cg = jax.lax.dot_general(
            oh, centers_ref[...], (((1,), (0,)), ((), ())),
            preferred_element_type=jnp.float32)
        diff = x - cg
        sq = diff * diff
        ones_col = jnp.ones((_D,), jnp.float32)
        rs = jax.lax.dot_general(
            sq, ones_col, (((1,), (0,)), ((), ())),
            preferred_element_type=jnp.float32)
        acc_ref[1] += jnp.sum(jnp.sqrt(rs))

        @pl.when(i == _NB - 1)
        def _fin():
            inter = acc_ref[0]
            intra = acc_ref[1]
            out_ref[0, 0] = jnp.where(inter > 0.0, intra / inter, intra)


def kernel(x, labels):
    labels3 = labels.astype(jnp.int32).reshape(_NB, 1, _BLK)
    out = pl.pallas_call(
        _loss_kernel,
        grid=(2, _NB),
        in_specs=[
            pl.BlockSpec((_BLK, _D), lambda p, i: (i, 0)),
            pl.BlockSpec((1, 1, _BLK), lambda p, i: (i, 0, 0)),
        ],
        out_shape=jax.ShapeDtypeStruct((1, 1), jnp.float32),
        out_specs=pl.BlockSpec(memory_space=pltpu.SMEM),
        scratch_shapes=[
            pltpu.VMEM((_KP, _D), jnp.float32),
            pltpu.VMEM((1, _KP), jnp.float32),
            pltpu.VMEM((_KP, _D), jnp.float32),
            pltpu.SMEM((2,), jnp.float32),
        ],
    )(x, labels3)
    return out[0, 0]


# speedup vs baseline: 7.2860x; 1.1429x over previous
import jax
import jax.numpy as jnp
from jax.experimental import pallas as pl
from jax.experimental.pallas import tpu as pltpu

_N = 320000
_D = 128
_K = 100
_KP = 128
_BLK = 3200
_NB = _N // _BLK


def _loss_kernel(x_ref, lab_ref, out_ref, sums_ref, counts_ref, centers_ref,
                 acc_ref):
    p = pl.program_id(0)
    i = pl.program_id(1)

    lab = lab_ref[0, 0, :]
    ids = jax.lax.broadcasted_iota(jnp.int16, (_BLK, _KP), 1)
    lab16 = lab.astype(jnp.int16)
    oh = jnp.where(lab16[:, None] == ids,
                   jnp.bfloat16(1), jnp.bfloat16(0))
    x = x_ref[...]

    @pl.when(jnp.logical_and(p == 0, i == 0))
    def _init():
        sums_ref[...] = jnp.zeros_like(sums_ref)
        counts_ref[...] = jnp.zeros_like(counts_ref)

    @pl.when(p == 0)
    def _accum():
        sums_ref[...] += jax.lax.dot_general(
            oh, x.astype(jnp.bfloat16), (((0,), (0,)), ((), ())),
            preferred_element_type=jnp.float32)
        ones_row = jnp.ones((1, _BLK), jnp.bfloat16)
        counts_ref[...] += jax.lax.dot_general(
            ones_row, oh, (((1,), (0,)), ((), ())),
            preferred_element_type=jnp.float32)

    @pl.when(jnp.logical_and(p == 1, i == 0))
    def _centers_and_inter():
        cnt = counts_ref[0, :]
        inv = jnp.where(cnt > 0.0, 1.0 / cnt, 0.0)
        centers = sums_ref[...] * inv[:, None]
        centers_ref[...] = centers
        g = jax.lax.dot_general(
            centers, centers, (((1,), (1,)), ((), ())),
            preferred_element_type=jnp.float32,
            precision=jax.lax.Precision.HIGHEST)
        n2 = jnp.sum(centers * centers, axis=1)
        d2 = n2[:, None] + n2[None, :] - 2.0 * g
        r = jax.lax.broadcasted_iota(jnp.int32, (_KP, _KP), 0)
        c = jax.lax.broadcasted_iota(jnp.int32, (_KP, _KP), 1)
        valid = jnp.logical_and(r < c, c < _K)
        d = jnp.sqrt(jnp.maximum(d2, 0.0))
        acc_ref[0] = jnp.sum(jnp.where(valid, d, 0.0))
        acc_ref[1] = 0.0

    @pl.when(p == 1)
    def _intra():
        cg = jax.lax.dot_general(
            oh, centers_ref[...].astype(jnp.bfloat16), (((1,), (0,)), ((), ())),
            preferred_element_type=jnp.float32)
        diff = x - cg
        sq_t = jnp.transpose(diff * diff)
        rs = jnp.sum(sq_t, axis=0)
        acc_ref[1] += jnp.sum(jnp.sqrt(rs))

        @pl.when(i == _NB - 1)
        def _fin():
            inter = acc_ref[0]
            intra = acc_ref[1]
            out_ref[0, 0] = jnp.where(inter > 0.0, intra / inter, intra)


def kernel(x, labels):
    labels3 = labels.astype(jnp.int32).reshape(_NB, 1, _BLK)
    out = pl.pallas_call(
        _loss_kernel,
        grid=(2, _NB),
        in_specs=[
            pl.BlockSpec((_BLK, _D), lambda p, i: (i, 0)),
            pl.BlockSpec((1, 1, _BLK), lambda p, i: (i, 0, 0)),
        ],
        out_shape=jax.ShapeDtypeStruct((1, 1), jnp.float32),
        out_specs=pl.BlockSpec(memory_space=pltpu.SMEM),
        scratch_shapes=[
            pltpu.VMEM((_KP, _D), jnp.float32),
            pltpu.VMEM((1, _KP), jnp.float32),
            pltpu.VMEM((_KP, _D), jnp.float32),
            pltpu.SMEM((2,), jnp.float32),
        ],
    )(x, labels3)
    return out[0, 0]


# speedup vs baseline: 9.6652x; 1.3265x over previous
import jax
import jax.numpy as jnp
from jax.experimental import pallas as pl
from jax.experimental.pallas import tpu as pltpu

_N = 320000
_D = 128
_K = 100
_KP = 128
_BLK = 6400
_NB = _N // _BLK


def _loss_kernel(x_ref, lab_ref, out_ref, sums_ref, counts_ref, centers_ref,
                 acc_ref):
    p = pl.program_id(0)
    i = pl.program_id(1)

    lab = lab_ref[0, 0, :]
    ids = jax.lax.broadcasted_iota(jnp.int16, (_BLK, _KP), 1)
    lab16 = lab.astype(jnp.int16)
    oh = jnp.where(lab16[:, None] == ids,
                   jnp.bfloat16(1), jnp.bfloat16(0))
    x = x_ref[...]

    @pl.when(jnp.logical_and(p == 0, i == 0))
    def _init():
        sums_ref[...] = jnp.zeros_like(sums_ref)
        counts_ref[...] = jnp.zeros_like(counts_ref)

    @pl.when(p == 0)
    def _accum():
        sums_ref[...] += jax.lax.dot_general(
            oh, x.astype(jnp.bfloat16), (((0,), (0,)), ((), ())),
            preferred_element_type=jnp.float32)
        ones_row = jnp.ones((1, _BLK), jnp.bfloat16)
        counts_ref[...] += jax.lax.dot_general(
            ones_row, oh, (((1,), (0,)), ((), ())),
            preferred_element_type=jnp.float32)

    @pl.when(jnp.logical_and(p == 1, i == 0))
    def _centers_and_inter():
        cnt = counts_ref[0, :]
        inv = jnp.where(cnt > 0.0, 1.0 / cnt, 0.0)
        centers = sums_ref[...] * inv[:, None]
        centers_ref[...] = centers
        g = jax.lax.dot_general(
            centers, centers, (((1,), (1,)), ((), ())),
            preferred_element_type=jnp.float32,
            precision=jax.lax.Precision.HIGHEST)
        n2 = jnp.sum(centers * centers, axis=1)
        d2 = n2[:, None] + n2[None, :] - 2.0 * g
        r = jax.lax.broadcasted_iota(jnp.int32, (_KP, _KP), 0)
        c = jax.lax.broadcasted_iota(jnp.int32, (_KP, _KP), 1)
        valid = jnp.logical_and(r < c, c < _K)
        d = jnp.sqrt(jnp.maximum(d2, 0.0))
        acc_ref[0] = jnp.sum(jnp.where(valid, d, 0.0))
        acc_ref[1] = 0.0

    @pl.when(p == 1)
    def _intra():
        cg = jax.lax.dot_general(
            oh, centers_ref[...].astype(jnp.bfloat16), (((1,), (0,)), ((), ())),
            preferred_element_type=jnp.float32)
        diff = x - cg
        sq_t = jnp.transpose(diff * diff)
        rs = jnp.sum(sq_t, axis=0)
        acc_ref[1] += jnp.sum(jnp.sqrt(rs))

        @pl.when(i == _NB - 1)
        def _fin():
            inter = acc_ref[0]
            intra = acc_ref[1]
            out_ref[0, 0] = jnp.where(inter > 0.0, intra / inter, intra)


def kernel(x, labels):
    labels3 = labels.astype(jnp.int32).reshape(_NB, 1, _BLK)
    out = pl.pallas_call(
        _loss_kernel,
        grid=(2, _NB),
        in_specs=[
            pl.BlockSpec((_BLK, _D), lambda p, i: (i, 0)),
            pl.BlockSpec((1, 1, _BLK), lambda p, i: (i, 0, 0)),
        ],
        out_shape=jax.ShapeDtypeStruct((1, 1), jnp.float32),
        out_specs=pl.BlockSpec(memory_space=pltpu.SMEM),
        scratch_shapes=[
            pltpu.VMEM((_KP, _D), jnp.float32),
            pltpu.VMEM((1, _KP), jnp.float32),
            pltpu.VMEM((_KP, _D), jnp.float32),
            pltpu.SMEM((2,), jnp.float32),
        ],
    )(x, labels3)
    return out[0, 0]


# speedup vs baseline: 11.1930x; 1.1581x over previous
import jax
import jax.numpy as jnp
from jax.experimental import pallas as pl
from jax.experimental.pallas import tpu as pltpu

_N = 320000
_D = 128
_K = 100
_KP = 128
_BLK = 12800
_NB = _N // _BLK


def _loss_kernel(x_ref, lab_ref, out_ref, sums_ref, counts_ref, centers_ref,
                 acc_ref):
    p = pl.program_id(0)
    i = pl.program_id(1)

    lab = lab_ref[0, 0, :]
    ids = jax.lax.broadcasted_iota(jnp.int16, (_BLK, _KP), 1)
    lab16 = lab.astype(jnp.int16)
    oh = jnp.where(lab16[:, None] == ids,
                   jnp.bfloat16(1), jnp.bfloat16(0))
    x = x_ref[...]

    @pl.when(jnp.logical_and(p == 0, i == 0))
    def _init():
        sums_ref[...] = jnp.zeros_like(sums_ref)
        counts_ref[...] = jnp.zeros_like(counts_ref)

    @pl.when(p == 0)
    def _accum():
        sums_ref[...] += jax.lax.dot_general(
            oh, x.astype(jnp.bfloat16), (((0,), (0,)), ((), ())),
            preferred_element_type=jnp.float32)
        ones_row = jnp.ones((1, _BLK), jnp.bfloat16)
        counts_ref[...] += jax.lax.dot_general(
            ones_row, oh, (((1,), (0,)), ((), ())),
            preferred_element_type=jnp.float32)

    @pl.when(jnp.logical_and(p == 1, i == 0))
    def _centers_and_inter():
        cnt = counts_ref[0, :]
        inv = jnp.where(cnt > 0.0, 1.0 / cnt, 0.0)
        centers = sums_ref[...] * inv[:, None]
        centers_ref[...] = centers
        g = jax.lax.dot_general(
            centers, centers, (((1,), (1,)), ((), ())),
            preferred_element_type=jnp.float32,
            precision=jax.lax.Precision.HIGHEST)
        n2 = jnp.sum(centers * centers, axis=1)
        d2 = n2[:, None] + n2[None, :] - 2.0 * g
        r = jax.lax.broadcasted_iota(jnp.int32, (_KP, _KP), 0)
        c = jax.lax.broadcasted_iota(jnp.int32, (_KP, _KP), 1)
        valid = jnp.logical_and(r < c, c < _K)
        d = jnp.sqrt(jnp.maximum(d2, 0.0))
        acc_ref[0] = jnp.sum(jnp.where(valid, d, 0.0))
        acc_ref[1] = 0.0

    @pl.when(p == 1)
    def _intra():
        cg = jax.lax.dot_general(
            oh, centers_ref[...].astype(jnp.bfloat16), (((1,), (0,)), ((), ())),
            preferred_element_type=jnp.float32)
        diff = x - cg
        sq_t = jnp.transpose(diff * diff)
        rs = jnp.sum(sq_t, axis=0)
        acc_ref[1] += jnp.sum(jnp.sqrt(rs))

        @pl.when(i == _NB - 1)
        def _fin():
            inter = acc_ref[0]
            intra = acc_ref[1]
            out_ref[0, 0] = jnp.where(inter > 0.0, intra / inter, intra)


def kernel(x, labels):
    labels3 = labels.astype(jnp.int32).reshape(_NB, 1, _BLK)
    out = pl.pallas_call(
        _loss_kernel,
        grid=(2, _NB),
        in_specs=[
            pl.BlockSpec((_BLK, _D), lambda p, i: (i, 0)),
            pl.BlockSpec((1, 1, _BLK), lambda p, i: (i, 0, 0)),
        ],
        out_shape=jax.ShapeDtypeStruct((1, 1), jnp.float32),
        out_specs=pl.BlockSpec(memory_space=pltpu.SMEM),
        scratch_shapes=[
            pltpu.VMEM((_KP, _D), jnp.float32),
            pltpu.VMEM((1, _KP), jnp.float32),
            pltpu.VMEM((_KP, _D), jnp.float32),
            pltpu.SMEM((2,), jnp.float32),
        ],
    )(x, labels3)
    return out[0, 0]


# speedup vs baseline: 11.4804x; 1.0257x over previous
import jax
import jax.numpy as jnp
from jax.experimental import pallas as pl
from jax.experimental.pallas import tpu as pltpu

_N = 320000
_D = 128
_K = 100
_KP = 128
_BLK = 16000
_NB = _N // _BLK


def _loss_kernel(x_ref, lab_ref, out_ref, sums_ref, counts_ref, centers_ref,
                 acc_ref):
    p = pl.program_id(0)
    i = pl.program_id(1)

    lab = lab_ref[0, 0, :]
    ids = jax.lax.broadcasted_iota(jnp.int16, (_BLK, _KP), 1)
    lab16 = lab.astype(jnp.int16)
    oh = jnp.where(lab16[:, None] == ids,
                   jnp.bfloat16(1), jnp.bfloat16(0))
    x = x_ref[...]

    @pl.when(jnp.logical_and(p == 0, i == 0))
    def _init():
        sums_ref[...] = jnp.zeros_like(sums_ref)
        counts_ref[...] = jnp.zeros_like(counts_ref)

    @pl.when(p == 0)
    def _accum():
        sums_ref[...] += jax.lax.dot_general(
            oh, x.astype(jnp.bfloat16), (((0,), (0,)), ((), ())),
            preferred_element_type=jnp.float32)
        ones_row = jnp.ones((1, _BLK), jnp.bfloat16)
        counts_ref[...] += jax.lax.dot_general(
            ones_row, oh, (((1,), (0,)), ((), ())),
            preferred_element_type=jnp.float32)

    @pl.when(jnp.logical_and(p == 1, i == 0))
    def _centers_and_inter():
        cnt = counts_ref[0, :]
        inv = jnp.where(cnt > 0.0, 1.0 / cnt, 0.0)
        centers = sums_ref[...] * inv[:, None]
        centers_ref[...] = centers
        g = jax.lax.dot_general(
            centers, centers, (((1,), (1,)), ((), ())),
            preferred_element_type=jnp.float32,
            precision=jax.lax.Precision.HIGHEST)
        n2 = jnp.sum(centers * centers, axis=1)
        d2 = n2[:, None] + n2[None, :] - 2.0 * g
        r = jax.lax.broadcasted_iota(jnp.int32, (_KP, _KP), 0)
        c = jax.lax.broadcasted_iota(jnp.int32, (_KP, _KP), 1)
        valid = jnp.logical_and(r < c, c < _K)
        d = jnp.sqrt(jnp.maximum(d2, 0.0))
        acc_ref[0] = jnp.sum(jnp.where(valid, d, 0.0))
        acc_ref[1] = 0.0

    @pl.when(p == 1)
    def _intra():
        cg = jax.lax.dot_general(
            oh, centers_ref[...].astype(jnp.bfloat16), (((1,), (0,)), ((), ())),
            preferred_element_type=jnp.float32)
        diff = x - cg
        sq_t = jnp.transpose(diff * diff)
        rs = jnp.sum(sq_t, axis=0)
        acc_ref[1] += jnp.sum(jnp.sqrt(rs))

        @pl.when(i == _NB - 1)
        def _fin():
            inter = acc_ref[0]
            intra = acc_ref[1]
            out_ref[0, 0] = jnp.where(inter > 0.0, intra / inter, intra)


def kernel(x, labels):
    labels3 = labels.astype(jnp.int32).reshape(_NB, 1, _BLK)
    out = pl.pallas_call(
        _loss_kernel,
        grid=(2, _NB),
        in_specs=[
            pl.BlockSpec((_BLK, _D), lambda p, i: (i, 0)),
            pl.BlockSpec((1, 1, _BLK), lambda p, i: (i, 0, 0)),
        ],
        out_shape=jax.ShapeDtypeStruct((1, 1), jnp.float32),
        out_specs=pl.BlockSpec(memory_space=pltpu.SMEM),
        scratch_shapes=[
            pltpu.VMEM((_KP, _D), jnp.float32),
            pltpu.VMEM((1, _KP), jnp.float32),
            pltpu.VMEM((_KP, _D), jnp.float32),
            pltpu.SMEM((2,), jnp.float32),
        ],
    )(x, labels3)
    return out[0, 0]


# speedup vs baseline: 11.8060x; 1.0284x over previous
import jax
import jax.numpy as jnp
from jax.experimental import pallas as pl
from jax.experimental.pallas import tpu as pltpu

_N = 320000
_D = 128
_K = 100
_KP = 128
_BLK = 16000
_NB = _N // _BLK


def _loss_kernel(x_ref, lab_ref, out_ref, sums_ref, counts_ref, centers_ref,
                 acc_ref):
    p = pl.program_id(0)
    i = pl.program_id(1)

    lab = lab_ref[0, 0, :]
    ids = jax.lax.broadcasted_iota(jnp.int16, (_BLK, _KP), 1)
    oh = jnp.where(lab.astype(jnp.int16)[:, None] == ids,
                   jnp.bfloat16(1), jnp.bfloat16(0))
    x_bf = x_ref[...].astype(jnp.bfloat16)

    @pl.when(jnp.logical_and(p == 0, i == 0))
    def _init():
        sums_ref[...] = jnp.zeros_like(sums_ref)
        counts_ref[...] = jnp.zeros_like(counts_ref)

    @pl.when(p == 0)
    def _accum():
        sums_ref[...] += jax.lax.dot_general(
            oh, x_bf, (((0,), (0,)), ((), ())),
            preferred_element_type=jnp.float32)
        ones_row = jnp.ones((1, _BLK), jnp.bfloat16)
        counts_ref[...] += jax.lax.dot_general(
            ones_row, oh, (((1,), (0,)), ((), ())),
            preferred_element_type=jnp.float32)

    @pl.when(jnp.logical_and(p == 1, i == 0))
    def _centers_and_inter():
        cnt = counts_ref[0, :]
        inv = jnp.where(cnt > 0.0, 1.0 / cnt, 0.0)
        centers = sums_ref[...] * inv[:, None]
        centers_ref[...] = centers
        g = jax.lax.dot_general(
            centers, centers, (((1,), (1,)), ((), ())),
            preferred_element_type=jnp.float32,
            precision=jax.lax.Precision.HIGHEST)
        n2 = jnp.sum(centers * centers, axis=1)
        d2 = n2[:, None] + n2[None, :] - 2.0 * g
        r = jax.lax.broadcasted_iota(jnp.int32, (_KP, _KP), 0)
        c = jax.lax.broadcasted_iota(jnp.int32, (_KP, _KP), 1)
        valid = jnp.logical_and(r < c, c < _K)
        d = jnp.sqrt(jnp.maximum(d2, 0.0))
        acc_ref[0] = jnp.sum(jnp.where(valid, d, 0.0))
        acc_ref[1] = 0.0

    @pl.when(p == 1)
    def _intra():
        centers_bf = centers_ref[...].astype(jnp.bfloat16)
        cg = jax.lax.dot_general(
            oh, centers_bf, (((1,), (0,)), ((), ())),
            preferred_element_type=jnp.float32).astype(jnp.bfloat16)
        diff = x_bf - cg
        sq = diff * diff
        ones_row = jnp.ones((1, _D), jnp.bfloat16)
        rs = jax.lax.dot_general(
            ones_row, sq, (((1,), (1,)), ((), ())),
            preferred_element_type=jnp.float32)[0]
        acc_ref[1] += jnp.sum(jnp.sqrt(rs))

        @pl.when(i == _NB - 1)
        def _fin():
            inter = acc_ref[0]
            intra = acc_ref[1]
            out_ref[0, 0] = jnp.where(inter > 0.0, intra / inter, intra)


def kernel(x, labels):
    labels3 = labels.astype(jnp.int32).reshape(_NB, 1, _BLK)
    out = pl.pallas_call(
        _loss_kernel,
        grid=(2, _NB),
        in_specs=[
            pl.BlockSpec((_BLK, _D), lambda p, i: (i, 0)),
            pl.BlockSpec((1, 1, _BLK), lambda p, i: (i, 0, 0)),
        ],
        out_shape=jax.ShapeDtypeStruct((1, 1), jnp.float32),
        out_specs=pl.BlockSpec(memory_space=pltpu.SMEM),
        scratch_shapes=[
            pltpu.VMEM((_KP, _D), jnp.float32),
            pltpu.VMEM((1, _KP), jnp.float32),
            pltpu.VMEM((_KP, _D), jnp.float32),
            pltpu.SMEM((2,), jnp.float32),
        ],
    )(x, labels3)
    return out[0, 0]
